# baseline (speedup 1.0000x reference)
import functools

import jax
import jax.numpy as jnp
from jax import lax
from jax.experimental import pallas as pl
from jax.experimental.pallas import tpu as pltpu
from jax.experimental.pallas import tpu_sc as plsc

B, L = 4096, 200
EMB = 14
HID = 16
N = B * L

_info = plsc.get_sparse_core_info()
NC, NS, LANES = _info.num_cores, _info.num_subcores, _info.num_lanes
NW = NC * NS
PER_W = N // NW
BLK = 1024
BLK2 = BLK * 2
NBLK = PER_W // BLK
NSTREAM = BLK // 128

_mesh = plsc.VectorSubcoreMesh(core_axis_name="c", subcore_axis_name="s")


@functools.partial(
    pl.kernel,
    mesh=_mesh,
    out_type=jax.ShapeDtypeStruct((N, HID), jnp.float32),
    scratch_types=[
        pltpu.VMEM((2 * BLK2,), jnp.float32),
        pltpu.VMEM((2 * BLK,), jnp.int32),
        pltpu.VMEM((2 * BLK,), jnp.float32),
        pltpu.VMEM((2 * BLK, HID), jnp.float32),
        pltpu.SemaphoreType.DMA,
        pltpu.SemaphoreType.DMA,
    ],
    compiler_params=pltpu.CompilerParams(
        needs_layout_passes=False,
        use_tc_tiling_on_sc=False,
    ),
)
def _sc_embed(table_h, x_h, out_h, x_v, idx_v, dur_v, out_v, sem_g, sem_o):
    wid = lax.axis_index("s") * NC + lax.axis_index("c")
    w_base = wid * PER_W
    lane = lax.iota(jnp.int32, LANES)
    rr_off = lane >> 1
    c_fix = (lane & 1) + EMB

    def stage_extract(g, s):
        base = w_base + g * BLK
        pltpu.sync_copy(x_h.at[pl.ds(base * 2, BLK2)], x_v.at[pl.ds(s * BLK2, BLK2)])

        def extract_body(j, c):
            pos = s * BLK2 + (j * LANES + lane) * 2
            fidx = plsc.load_gather(x_v, [pos])
            fdur = plsc.load_gather(x_v, [pos + 1])
            idx_v[pl.ds(s * BLK + j * LANES, LANES)] = fidx.astype(jnp.int32)
            dur_v[pl.ds(s * BLK + j * LANES, LANES)] = fdur
            return c

        lax.fori_loop(0, BLK // LANES, extract_body, 0)

    def fire_gathers(s):
        for j in range(NSTREAM):
            pltpu.async_copy(
                table_h.at[idx_v.at[pl.ds(s * BLK + j * 128, 128)]],
                out_v.at[pl.ds(s * BLK + j * 128, 128)],
                sem_g,
            )

    def drain_gathers(s):
        for j in range(NSTREAM):
            pltpu.make_async_copy(
                table_h.at[idx_v.at[pl.ds(s * BLK + j * 128, 128)]],
                out_v.at[pl.ds(s * BLK + j * 128, 128)],
                sem_g,
            ).wait()

    def out_desc(g, s):
        base = w_base + g * BLK
        return pltpu.make_async_copy(
            out_v.at[pl.ds(s * BLK, BLK)], out_h.at[pl.ds(base, BLK)], sem_o
        )

    stage_extract(0, 0)
    fire_gathers(0)

    def block_body(g, carry):
        s = lax.rem(g, 2)
        s1 = 1 - s

        @pl.when(g + 1 < NBLK)
        def _():
            stage_extract(g + 1, s1)

            @pl.when(g >= 1)
            def _():
                out_desc(g - 1, s1).wait()

            fire_gathers(s1)

        drain_gathers(s)

        def fix_body(j, c):
            r_idx = s * BLK + j * 8 + rr_off
            val = plsc.load_gather(dur_v, [r_idx])
            plsc.store_scatter(out_v, [r_idx, c_fix], val)
            return c

        lax.fori_loop(0, BLK // 8, fix_body, 0)

        out_desc(g, s).start()
        return carry

    lax.fori_loop(0, NBLK, block_body, 0)

    out_desc(NBLK - 2, lax.rem(NBLK - 2, 2)).wait()
    out_desc(NBLK - 1, lax.rem(NBLK - 1, 2)).wait()


def kernel(x, table):
    table16 = jnp.pad(table, ((0, 0), (0, HID - EMB)))
    x_flat = x.reshape(N * 2)
    out = _sc_embed(table16, x_flat)
    return out.reshape(B, L, HID)
